# baseline (speedup 1.0000x reference)
import dataclasses
import functools

import jax
import jax.numpy as jnp
from jax import lax
from jax.experimental import pallas as pl
from jax.experimental.pallas import tpu as pltpu
from jax.experimental.pallas import tpu_sc as plsc

_N = 10000
_NH = 8
_D = 64
_E = 160000
_EB = 2048
_EPAD = 163840
_S = 2048
_T = 2000
_HI = lax.Precision.HIGHEST
_HG = lax.Precision.HIGH
_NW = 32


def _tf_body(f_ref, w_ref, b_ref, o_ref):
    o_ref[...] = (
        jnp.dot(f_ref[...], w_ref[0], precision=_HI,
                preferred_element_type=jnp.float32)
        + b_ref[0]
    )


def _type_transform(featcat, wstack, bstack):
    def _wmap(i):
        return (jnp.where(i < 2, 0, jnp.where(i < 6, 1, 2)), 0, 0)

    return pl.pallas_call(
        _tf_body,
        grid=(10,),
        in_specs=[
            pl.BlockSpec((1000, 128), lambda i: (i, 0)),
            pl.BlockSpec((1, 128, _D), _wmap),
            pl.BlockSpec((1, 1, _D), _wmap),
        ],
        out_specs=pl.BlockSpec((1000, _D), lambda i: (i, 0)),
        out_shape=jax.ShapeDtypeStruct((_N, _D), jnp.float32),
    )(featcat, wstack, bstack)


def _hgnn_body(emb_ref, tfb_ref, emb0t_ref, wac_ref, wact_ref, fsrc_ref,
               o_ref):
    i = pl.program_id(0)

    @pl.when(i < 2)
    def _copy():
        o_ref[...] = tfb_ref[...]

    @pl.when(i >= 2)
    def _complete():
        embb = emb_ref[...].astype(jnp.bfloat16)
        fsrc = fsrc_ref[...].astype(jnp.bfloat16)
        emb0t = emb0t_ref[...].astype(jnp.bfloat16)

        def head(h, acc):
            h1 = jax.nn.sigmoid(
                jnp.dot(embb, wac_ref[h].astype(jnp.bfloat16),
                        preferred_element_type=jnp.float32))
            h2t = jax.nn.sigmoid(
                jnp.dot(wact_ref[h].astype(jnp.bfloat16), emb0t,
                        preferred_element_type=jnp.float32))
            h1hi = h1.astype(jnp.bfloat16)
            h1lo = (h1 - h1hi.astype(jnp.float32)).astype(jnp.bfloat16)
            h2tb = h2t.astype(jnp.bfloat16)
            lg = (jnp.dot(h1hi, h2tb, preferred_element_type=jnp.float32)
                  + jnp.dot(h1lo, h2tb, preferred_element_type=jnp.float32))
            lg = lg - jnp.max(lg, axis=-1, keepdims=True)
            el = jnp.exp(lg)
            sm = el / jnp.sum(el, axis=-1, keepdims=True)
            return acc + jnp.dot(sm.astype(jnp.bfloat16), fsrc,
                                 preferred_element_type=jnp.float32)

        acc = lax.fori_loop(0, _NH, head, jnp.zeros((1000, _D), jnp.float32))
        o_ref[...] = tfb_ref[...] + acc * (1.0 / _NH)


def _hgnn_complete(emb, tf_base, emb0t, wac, wact):
    return pl.pallas_call(
        _hgnn_body,
        grid=(10,),
        in_specs=[
            pl.BlockSpec((1000, _D), lambda i: (i, 0)),
            pl.BlockSpec((1000, _D), lambda i: (i, 0)),
            pl.BlockSpec((_D, _T), lambda i: (0, 0)),
            pl.BlockSpec((_NH, _D, 128), lambda i: (0, 0, 0)),
            pl.BlockSpec((_NH, 128, _D), lambda i: (0, 0, 0)),
            pl.BlockSpec((_T, _D), lambda i: (0, 0)),
        ],
        out_specs=pl.BlockSpec((1000, _D), lambda i: (i, 0)),
        out_shape=jax.ShapeDtypeStruct((_N, _D), jnp.float32),
    )(emb, tf_base, emb0t, wac, wact, tf_base)


def _tpad_body(tcol_ref, trow_ref, o_ref):
    i = pl.program_id(0)
    tcol = tcol_ref[...]
    trow = trow_ref[...]
    iota_t = lax.broadcasted_iota(jnp.int32, (512, _S), 1)
    cand = jnp.where(tcol == trow, iota_t, -1)
    last = jnp.max(cand, axis=1, keepdims=True)
    iota_s = lax.broadcasted_iota(jnp.int32, (512, 1), 0) + i * 512
    o_ref[...] = jnp.where(last == iota_s, tcol, -1)


def _make_tpad(tcol, trow):
    return pl.pallas_call(
        _tpad_body,
        grid=(4,),
        in_specs=[
            pl.BlockSpec((512, 1), lambda i: (i, 0)),
            pl.BlockSpec((1, _S), lambda i: (0, 0)),
        ],
        out_specs=pl.BlockSpec((512, 1), lambda i: (i, 0)),
        out_shape=jax.ShapeDtypeStruct((_S, 1), jnp.int32),
    )(tcol, trow)


def _gather_rows(table, idx_flat):
    w = 512
    nblk = _EPAD // (_NW * w)
    mesh = plsc.VectorSubcoreMesh(core_axis_name="c", subcore_axis_name="s")

    @functools.partial(
        pl.kernel,
        mesh=mesh,
        compiler_params=pltpu.CompilerParams(use_tc_tiling_on_sc=False),
        out_type=jax.ShapeDtypeStruct((2 * _EPAD, 2 * _D), jnp.float32),
        scratch_types=[
            pltpu.VMEM((3, w), jnp.int32),
            pltpu.VMEM((w, _D), jnp.float32),
            pltpu.VMEM((w, _D), jnp.float32),
            pltpu.VMEM((w, _D), jnp.float32),
            pltpu.SemaphoreType.DMA,
            pltpu.SemaphoreType.DMA,
            pltpu.SemaphoreType.DMA,
            pltpu.SemaphoreType.DMA,
        ],
    )
    def k(table_hbm, idx_hbm, out_hbm, idx_v, b0, b1, b2, g0s, g1s, g2s, ws):
        wid = lax.axis_index("c") * 16 + lax.axis_index("s")
        base = wid * (nblk * w)

        @pl.loop(0, nblk)
        def _(j):
            e0 = base + j * w
            pltpu.sync_copy(idx_hbm.at[pl.ds(e0, w)], idx_v.at[0])
            ga = pltpu.async_copy(table_hbm.at[idx_v.at[0]], b0, g0s)
            pltpu.sync_copy(idx_hbm.at[pl.ds(_EPAD + e0, w)], idx_v.at[1])
            gb = pltpu.async_copy(table_hbm.at[idx_v.at[1]], b1, g1s)
            pltpu.sync_copy(idx_hbm.at[pl.ds(2 * _EPAD + e0, w)], idx_v.at[2])
            gc = pltpu.async_copy(table_hbm.at[idx_v.at[2]], b2, g2s)
            ga.wait()
            w0 = pltpu.async_copy(b0, out_hbm.at[pl.ds(e0, w), pl.ds(0, _D)],
                                  ws)
            gb.wait()
            w1 = pltpu.async_copy(b1, out_hbm.at[pl.ds(e0, w), pl.ds(_D, _D)],
                                  ws)
            gc.wait()
            w2 = pltpu.async_copy(
                b2, out_hbm.at[pl.ds(_EPAD + e0, w), pl.ds(0, _D)], ws)
            w0.wait()
            w1.wait()
            w2.wait()

    return k(table, idx_flat)


def _seg_body(g01_ref, g2_ref, dst_ref, tpad_ref, attn_ref, acc_ref):
    i = pl.program_id(0)

    @pl.when(i == 0)
    def _init():
        acc_ref[...] = jnp.zeros_like(acc_ref)

    g01 = g01_ref[...]
    hidden = (g01[:, :_D] + g01[:, _D:] + g2_ref[:, :_D]) * (1.0 / 3.0)
    a = jnp.dot(hidden, attn_ref[...], precision=_HI)
    a = jnp.where(a >= 0, a, 0.2 * a)
    ea = jnp.exp(a)
    rhs = jnp.concatenate(
        [hidden * ea[:, h:h + 1] for h in range(_NH)] + [ea], axis=1
    )
    oh = (tpad_ref[...] == dst_ref[0]).astype(jnp.bfloat16)
    acc_ref[...] += jnp.dot(oh, rhs.astype(jnp.bfloat16),
                            preferred_element_type=jnp.float32)


def _segment_accumulate(gflat, dst3d, tpad, attn_w):
    nsteps = _EPAD // _EB
    nb = _EPAD // _EB
    return pl.pallas_call(
        _seg_body,
        grid=(nsteps,),
        in_specs=[
            pl.BlockSpec((_EB, 2 * _D), lambda i: (i, 0)),
            pl.BlockSpec((_EB, 2 * _D), lambda i: (nb + i, 0)),
            pl.BlockSpec((1, 1, _EB), lambda i: (i, 0, 0)),
            pl.BlockSpec((_S, 1), lambda i: (0, 0)),
            pl.BlockSpec((_D, _NH), lambda i: (0, 0)),
        ],
        out_specs=pl.BlockSpec((_S, _NH * _D + _NH), lambda i: (0, 0)),
        out_shape=jax.ShapeDtypeStruct((_S, _NH * _D + _NH), jnp.float32),
    )(gflat, gflat, dst3d, tpad, attn_w)


def _slotgather_body(acc_ref, tpadrow_ref, tcol_ref, o_ref):
    acc = acc_ref[...]
    div = jnp.concatenate(
        [acc[:, h * _D:(h + 1) * _D]
         / (acc[:, _NH * _D + h:_NH * _D + h + 1] + 1e-9)
         for h in range(_NH)], axis=1)
    m = (tcol_ref[...] == tpadrow_ref[...]).astype(jnp.float32)
    ret = jnp.dot(m, div, precision=_HI)[:_T]
    o_ref[...] = jnp.where(ret > 0, ret,
                           jnp.exp(jnp.minimum(ret, 0.0)) - 1.0)


def _slotgather(acc, tpadrow, tcol3):
    w = _NH * _D + _NH
    full = lambda shape: pl.BlockSpec(shape, lambda: tuple(0 for _ in shape))
    return pl.pallas_call(
        _slotgather_body,
        in_specs=[full((_S, w)), full((1, _S)), full((_S, 1))],
        out_specs=full((_T, _NH * _D)),
        out_shape=jax.ShapeDtypeStruct((_T, _NH * _D), jnp.float32),
    )(acc, tpadrow, tcol3)


def _fin_body(o0_ref, o1_ref, fc1w_ref, fc1b_ref,
              fc2w_ref, fc2b_ref, fcw_ref, fcb_ref, logits_ref, h_ref):
    outs = [o0_ref[...], o1_ref[...]]
    scores = []
    for o in outs:
        t = jnp.tanh(jnp.dot(o, fc1w_ref[...], precision=_HI) + fc1b_ref[...])
        sc = jnp.dot(t, fc2w_ref[...], precision=_HI) + fc2b_ref[...]
        scores.append(jnp.mean(sc))
    mx = jnp.maximum(scores[0], scores[1])
    e0 = jnp.exp(scores[0] - mx)
    e1 = jnp.exp(scores[1] - mx)
    b0 = e0 / (e0 + e1)
    b1 = e1 / (e0 + e1)
    h = b0 * outs[0] + b1 * outs[1]
    h_ref[...] = h
    logits_ref[...] = jnp.dot(h, fcw_ref[...], precision=_HI) + fcb_ref[...]


def _finalize(out0, out1, fc1_w, fc1_b, fc2_w, fc2_b, fc_w, fc_b):
    full = lambda shape: pl.BlockSpec(shape, lambda: tuple(0 for _ in shape))
    return pl.pallas_call(
        _fin_body,
        in_specs=[
            full((_T, _NH * _D)), full((_T, _NH * _D)),
            full((_NH * _D, 128)), full((1, 128)), full((128, 1)),
            full((1, 1)), full((_NH * _D, 3)), full((1, 3)),
        ],
        out_specs=[full((_T, 3)), full((_T, _NH * _D))],
        out_shape=[
            jax.ShapeDtypeStruct((_T, 3), jnp.float32),
            jax.ShapeDtypeStruct((_T, _NH * _D), jnp.float32),
        ],
    )(out0, out1, fc1_w, fc1_b, fc2_w, fc2_b, fc_w, fc_b)


def _prep_layer_idx(mp_idx):
    idxt = jnp.transpose(mp_idx)
    idxt = jnp.pad(idxt, ((0, 0), (0, _EPAD - _E)))
    return idxt.reshape(-1)


def _prep_dst(mp_idx):
    dst = jnp.pad(mp_idx[:, 2], (0, _EPAD - _E), constant_values=-2)
    return dst.reshape(_EPAD // _EB, 1, _EB)


def kernel(feat0, feat1, feat2, emb, W0, b0, W1, b1, W2, b2, W_ac,
           attn0, attn1, fc1_W, fc1_b, fc2_W, fc2_b, fc_W, fc_b,
           type_mask, mask0, mask1, mask2, mp_idx0, mp_idx1, target_idx):
    featcat = jnp.concatenate([feat0, feat1, feat2], axis=0)
    wstack = jnp.stack([W0, W1, W2])
    bstack = jnp.stack([b0, b1, b2]).reshape(3, 1, _D)
    tf_base = _type_transform(featcat, wstack, bstack)

    emb0t = jnp.transpose(emb[:_T])
    wact = jnp.transpose(W_ac, (0, 2, 1))
    tf_fin = _hgnn_complete(emb, tf_base, emb0t, W_ac, wact)

    tcol = jnp.pad(target_idx, (0, _S - _T),
                   constant_values=-1).reshape(_S, 1)
    trow = jnp.pad(target_idx, (0, _S - _T),
                   constant_values=-2).reshape(1, _S)
    tpad = _make_tpad(tcol, trow)

    g0 = _gather_rows(tf_fin, _prep_layer_idx(mp_idx0))
    g1 = _gather_rows(tf_fin, _prep_layer_idx(mp_idx1))

    acc0 = _segment_accumulate(g0, _prep_dst(mp_idx0), tpad,
                               jnp.transpose(attn0))
    acc1 = _segment_accumulate(g1, _prep_dst(mp_idx1), tpad,
                               jnp.transpose(attn1))

    tcol3 = jnp.pad(target_idx, (0, _S - _T),
                    constant_values=-3).reshape(_S, 1)
    tpadrow = tpad.reshape(1, _S)
    out0 = _slotgather(acc0, tpadrow, tcol3)
    out1 = _slotgather(acc1, tpadrow, tcol3)
    logits, h = _finalize(
        out0, out1,
        fc1_W, fc1_b.reshape(1, 128), fc2_W, fc2_b.reshape(1, 1),
        fc_W, fc_b.reshape(1, 3))
    return (logits, h)


# speedup vs baseline: 1.0378x; 1.0378x over previous
import dataclasses
import functools

import jax
import jax.numpy as jnp
from jax import lax
from jax.experimental import pallas as pl
from jax.experimental.pallas import tpu as pltpu
from jax.experimental.pallas import tpu_sc as plsc

_N = 10000
_NH = 8
_D = 64
_E = 160000
_EB = 2048
_EPAD = 163840
_S = 2048
_T = 2000
_HI = lax.Precision.HIGHEST
_HG = lax.Precision.HIGH
_NW = 32


def _tf_body(f_ref, w_ref, b_ref, o_ref):
    o_ref[...] = (
        jnp.dot(f_ref[...], w_ref[0], precision=_HI,
                preferred_element_type=jnp.float32)
        + b_ref[0]
    )


def _type_transform(featcat, wstack, bstack):
    def _wmap(i):
        return (jnp.where(i < 2, 0, jnp.where(i < 6, 1, 2)), 0, 0)

    return pl.pallas_call(
        _tf_body,
        grid=(10,),
        in_specs=[
            pl.BlockSpec((1000, 128), lambda i: (i, 0)),
            pl.BlockSpec((1, 128, _D), _wmap),
            pl.BlockSpec((1, 1, _D), _wmap),
        ],
        out_specs=pl.BlockSpec((1000, _D), lambda i: (i, 0)),
        out_shape=jax.ShapeDtypeStruct((_N, _D), jnp.float32),
    )(featcat, wstack, bstack)


def _pack_bf16(v):
    xi = lax.bitcast_convert_type(v, jnp.int32)
    lo = lax.shift_right_logical(xi[:, :_D // 2] + 0x8000, 16)
    hi = (xi[:, _D // 2:] + 0x8000) & jnp.int32(-65536)
    return lo | hi


def _unpack_bf16(x):
    lo = lax.bitcast_convert_type(lax.shift_left(x, 16), jnp.float32)
    hi = lax.bitcast_convert_type(x & jnp.int32(-65536), jnp.float32)
    return lo, hi


def _hgnn_body(emb_ref, tfb_ref, emb0t_ref, wac_ref, wact_ref, fsrc_ref,
               o_ref):
    i = pl.program_id(0)

    @pl.when(i < 2)
    def _copy():
        o_ref[...] = _pack_bf16(tfb_ref[...])

    @pl.when(i >= 2)
    def _complete():
        embb = emb_ref[...].astype(jnp.bfloat16)
        fsrc = fsrc_ref[...].astype(jnp.bfloat16)
        emb0t = emb0t_ref[...].astype(jnp.bfloat16)

        def head(h, acc):
            h1 = jax.nn.sigmoid(
                jnp.dot(embb, wac_ref[h].astype(jnp.bfloat16),
                        preferred_element_type=jnp.float32))
            h2t = jax.nn.sigmoid(
                jnp.dot(wact_ref[h].astype(jnp.bfloat16), emb0t,
                        preferred_element_type=jnp.float32))
            h1hi = h1.astype(jnp.bfloat16)
            h1lo = (h1 - h1hi.astype(jnp.float32)).astype(jnp.bfloat16)
            h2tb = h2t.astype(jnp.bfloat16)
            lg = (jnp.dot(h1hi, h2tb, preferred_element_type=jnp.float32)
                  + jnp.dot(h1lo, h2tb, preferred_element_type=jnp.float32))
            lg = lg - jnp.max(lg, axis=-1, keepdims=True)
            el = jnp.exp(lg)
            sm = el / jnp.sum(el, axis=-1, keepdims=True)
            return acc + jnp.dot(sm.astype(jnp.bfloat16), fsrc,
                                 preferred_element_type=jnp.float32)

        acc = lax.fori_loop(0, _NH, head, jnp.zeros((1000, _D), jnp.float32))
        o_ref[...] = _pack_bf16(tfb_ref[...] + acc * (1.0 / _NH))


def _hgnn_complete(emb, tf_base, emb0t, wac, wact):
    return pl.pallas_call(
        _hgnn_body,
        grid=(10,),
        in_specs=[
            pl.BlockSpec((1000, _D), lambda i: (i, 0)),
            pl.BlockSpec((1000, _D), lambda i: (i, 0)),
            pl.BlockSpec((_D, _T), lambda i: (0, 0)),
            pl.BlockSpec((_NH, _D, 128), lambda i: (0, 0, 0)),
            pl.BlockSpec((_NH, 128, _D), lambda i: (0, 0, 0)),
            pl.BlockSpec((_T, _D), lambda i: (0, 0)),
        ],
        out_specs=pl.BlockSpec((1000, _D // 2), lambda i: (i, 0)),
        out_shape=jax.ShapeDtypeStruct((_N, _D // 2), jnp.int32),
    )(emb, tf_base, emb0t, wac, wact, tf_base)


def _tpad_body(tcol_ref, trow_ref, o_ref):
    i = pl.program_id(0)
    tcol = tcol_ref[...]
    trow = trow_ref[...]
    iota_t = lax.broadcasted_iota(jnp.int32, (512, _S), 1)
    cand = jnp.where(tcol == trow, iota_t, -1)
    last = jnp.max(cand, axis=1, keepdims=True)
    iota_s = lax.broadcasted_iota(jnp.int32, (512, 1), 0) + i * 512
    o_ref[...] = jnp.where(last == iota_s, tcol, -1)


def _make_tpad(tcol, trow):
    return pl.pallas_call(
        _tpad_body,
        grid=(4,),
        in_specs=[
            pl.BlockSpec((512, 1), lambda i: (i, 0)),
            pl.BlockSpec((1, _S), lambda i: (0, 0)),
        ],
        out_specs=pl.BlockSpec((512, 1), lambda i: (i, 0)),
        out_shape=jax.ShapeDtypeStruct((_S, 1), jnp.int32),
    )(tcol, trow)


def _gather_rows(table, idx_flat):
    w = 1024
    nblk = _EPAD // (_NW * w)
    hw = _D // 2
    mesh = plsc.VectorSubcoreMesh(core_axis_name="c", subcore_axis_name="s")

    @functools.partial(
        pl.kernel,
        mesh=mesh,
        compiler_params=pltpu.CompilerParams(use_tc_tiling_on_sc=False),
        out_type=jax.ShapeDtypeStruct((_EPAD, 2 * _D), jnp.int32),
        scratch_types=[
            pltpu.VMEM((3, w), jnp.int32),
            pltpu.VMEM((w, hw), jnp.int32),
            pltpu.VMEM((w, hw), jnp.int32),
            pltpu.VMEM((w, hw), jnp.int32),
            pltpu.SemaphoreType.DMA,
            pltpu.SemaphoreType.DMA,
            pltpu.SemaphoreType.DMA,
            pltpu.SemaphoreType.DMA,
        ],
    )
    def k(table_hbm, idx_hbm, out_hbm, idx_v, b0, b1, b2, g0s, g1s, g2s, ws):
        wid = lax.axis_index("c") * 16 + lax.axis_index("s")
        base = wid * (nblk * w)

        @pl.loop(0, nblk)
        def _(j):
            e0 = base + j * w
            pltpu.sync_copy(idx_hbm.at[pl.ds(e0, w)], idx_v.at[0])
            ga = pltpu.async_copy(table_hbm.at[idx_v.at[0]], b0, g0s)
            pltpu.sync_copy(idx_hbm.at[pl.ds(_EPAD + e0, w)], idx_v.at[1])
            gb = pltpu.async_copy(table_hbm.at[idx_v.at[1]], b1, g1s)
            pltpu.sync_copy(idx_hbm.at[pl.ds(2 * _EPAD + e0, w)], idx_v.at[2])
            gc = pltpu.async_copy(table_hbm.at[idx_v.at[2]], b2, g2s)
            ga.wait()
            w0 = pltpu.async_copy(b0, out_hbm.at[pl.ds(e0, w), pl.ds(0, hw)],
                                  ws)
            gb.wait()
            w1 = pltpu.async_copy(b1, out_hbm.at[pl.ds(e0, w), pl.ds(hw, hw)],
                                  ws)
            gc.wait()
            w2 = pltpu.async_copy(
                b2, out_hbm.at[pl.ds(e0, w), pl.ds(2 * hw, hw)], ws)
            w0.wait()
            w1.wait()
            w2.wait()

    return k(table, idx_flat)


def _seg_body(g_ref, dst_ref, tpad_ref, attn_ref, acc_ref):
    i = pl.program_id(0)
    hw = _D // 2

    @pl.when(i == 0)
    def _init():
        acc_ref[...] = jnp.zeros_like(acc_ref)

    g = g_ref[...]
    lo0, hi0 = _unpack_bf16(g[:, :hw])
    lo1, hi1 = _unpack_bf16(g[:, hw:2 * hw])
    lo2, hi2 = _unpack_bf16(g[:, 2 * hw:3 * hw])
    hidden = (jnp.concatenate([lo0 + lo1 + lo2, hi0 + hi1 + hi2], axis=1)
              * (1.0 / 3.0))
    a = jnp.dot(hidden, attn_ref[...], precision=_HI)
    a = jnp.where(a >= 0, a, 0.2 * a)
    ea = jnp.exp(a)
    rhs = jnp.concatenate(
        [hidden * ea[:, h:h + 1] for h in range(_NH)] + [ea], axis=1
    )
    oh = (tpad_ref[...] == dst_ref[0]).astype(jnp.bfloat16)
    acc_ref[...] += jnp.dot(oh, rhs.astype(jnp.bfloat16),
                            preferred_element_type=jnp.float32)


def _segment_accumulate(gflat, dst3d, tpad, attn_w):
    nsteps = _EPAD // _EB
    return pl.pallas_call(
        _seg_body,
        grid=(nsteps,),
        in_specs=[
            pl.BlockSpec((_EB, 2 * _D), lambda i: (i, 0)),
            pl.BlockSpec((1, 1, _EB), lambda i: (i, 0, 0)),
            pl.BlockSpec((_S, 1), lambda i: (0, 0)),
            pl.BlockSpec((_D, _NH), lambda i: (0, 0)),
        ],
        out_specs=pl.BlockSpec((_S, _NH * _D + _NH), lambda i: (0, 0)),
        out_shape=jax.ShapeDtypeStruct((_S, _NH * _D + _NH), jnp.float32),
    )(gflat, dst3d, tpad, attn_w)


def _slotgather_body(acc_ref, tpadrow_ref, tcol_ref, o_ref):
    acc = acc_ref[...]
    div = jnp.concatenate(
        [acc[:, h * _D:(h + 1) * _D]
         / (acc[:, _NH * _D + h:_NH * _D + h + 1] + 1e-9)
         for h in range(_NH)], axis=1)
    m = (tcol_ref[...] == tpadrow_ref[...]).astype(jnp.float32)
    ret = jnp.dot(m, div, precision=_HI)[:_T]
    o_ref[...] = jnp.where(ret > 0, ret,
                           jnp.exp(jnp.minimum(ret, 0.0)) - 1.0)


def _slotgather(acc, tpadrow, tcol3):
    w = _NH * _D + _NH
    full = lambda shape: pl.BlockSpec(shape, lambda: tuple(0 for _ in shape))
    return pl.pallas_call(
        _slotgather_body,
        in_specs=[full((_S, w)), full((1, _S)), full((_S, 1))],
        out_specs=full((_T, _NH * _D)),
        out_shape=jax.ShapeDtypeStruct((_T, _NH * _D), jnp.float32),
    )(acc, tpadrow, tcol3)


def _fin_body(o0_ref, o1_ref, fc1w_ref, fc1b_ref,
              fc2w_ref, fc2b_ref, fcw_ref, fcb_ref, logits_ref, h_ref):
    outs = [o0_ref[...], o1_ref[...]]
    scores = []
    for o in outs:
        t = jnp.tanh(jnp.dot(o, fc1w_ref[...], precision=_HI) + fc1b_ref[...])
        sc = jnp.dot(t, fc2w_ref[...], precision=_HI) + fc2b_ref[...]
        scores.append(jnp.mean(sc))
    mx = jnp.maximum(scores[0], scores[1])
    e0 = jnp.exp(scores[0] - mx)
    e1 = jnp.exp(scores[1] - mx)
    b0 = e0 / (e0 + e1)
    b1 = e1 / (e0 + e1)
    h = b0 * outs[0] + b1 * outs[1]
    h_ref[...] = h
    logits_ref[...] = jnp.dot(h, fcw_ref[...], precision=_HI) + fcb_ref[...]


def _finalize(out0, out1, fc1_w, fc1_b, fc2_w, fc2_b, fc_w, fc_b):
    full = lambda shape: pl.BlockSpec(shape, lambda: tuple(0 for _ in shape))
    return pl.pallas_call(
        _fin_body,
        in_specs=[
            full((_T, _NH * _D)), full((_T, _NH * _D)),
            full((_NH * _D, 128)), full((1, 128)), full((128, 1)),
            full((1, 1)), full((_NH * _D, 3)), full((1, 3)),
        ],
        out_specs=[full((_T, 3)), full((_T, _NH * _D))],
        out_shape=[
            jax.ShapeDtypeStruct((_T, 3), jnp.float32),
            jax.ShapeDtypeStruct((_T, _NH * _D), jnp.float32),
        ],
    )(out0, out1, fc1_w, fc1_b, fc2_w, fc2_b, fc_w, fc_b)


def _prep_layer_idx(mp_idx):
    idxt = jnp.transpose(mp_idx)
    idxt = jnp.pad(idxt, ((0, 0), (0, _EPAD - _E)))
    return idxt.reshape(-1)


def _prep_dst(mp_idx):
    dst = jnp.pad(mp_idx[:, 2], (0, _EPAD - _E), constant_values=-2)
    return dst.reshape(_EPAD // _EB, 1, _EB)


def kernel(feat0, feat1, feat2, emb, W0, b0, W1, b1, W2, b2, W_ac,
           attn0, attn1, fc1_W, fc1_b, fc2_W, fc2_b, fc_W, fc_b,
           type_mask, mask0, mask1, mask2, mp_idx0, mp_idx1, target_idx):
    featcat = jnp.concatenate([feat0, feat1, feat2], axis=0)
    wstack = jnp.stack([W0, W1, W2])
    bstack = jnp.stack([b0, b1, b2]).reshape(3, 1, _D)
    tf_base = _type_transform(featcat, wstack, bstack)

    emb0t = jnp.transpose(emb[:_T])
    wact = jnp.transpose(W_ac, (0, 2, 1))
    tf_fin = _hgnn_complete(emb, tf_base, emb0t, W_ac, wact)

    tcol = jnp.pad(target_idx, (0, _S - _T),
                   constant_values=-1).reshape(_S, 1)
    trow = jnp.pad(target_idx, (0, _S - _T),
                   constant_values=-2).reshape(1, _S)
    tpad = _make_tpad(tcol, trow)

    g0 = _gather_rows(tf_fin, _prep_layer_idx(mp_idx0))
    g1 = _gather_rows(tf_fin, _prep_layer_idx(mp_idx1))

    acc0 = _segment_accumulate(g0, _prep_dst(mp_idx0), tpad,
                               jnp.transpose(attn0))
    acc1 = _segment_accumulate(g1, _prep_dst(mp_idx1), tpad,
                               jnp.transpose(attn1))

    tcol3 = jnp.pad(target_idx, (0, _S - _T),
                    constant_values=-3).reshape(_S, 1)
    tpadrow = tpad.reshape(1, _S)
    out0 = _slotgather(acc0, tpadrow, tcol3)
    out1 = _slotgather(acc1, tpadrow, tcol3)
    logits, h = _finalize(
        out0, out1,
        fc1_W, fc1_b.reshape(1, 128), fc2_W, fc2_b.reshape(1, 1),
        fc_W, fc_b.reshape(1, 3))
    return (logits, h)
